# Initial kernel scaffold; baseline (speedup 1.0000x reference)
import jax
import jax.numpy as jnp
from jax.experimental import pallas as pl


def kernel(point_cloud_features):
    raise NotImplementedError("write your pallas kernel here")


# speedup vs baseline: 4.2548x; 4.2548x over previous
import functools

import jax
import jax.numpy as jnp
from jax import lax
from jax.experimental import pallas as pl
from jax.experimental.pallas import tpu as pltpu
from jax.experimental.pallas import tpu_sc as plsc

B = 4
N = 50000
C = 101
NX, NY, NZ = 40, 80, 80
VSIZE = 0.05
MAXV = 2000
MAXP = 25
SENTINEL = NX * NY * NZ

CP = 128
NCH = N // CP
CPT = (NCH + 15) // 16
TAIL = N - NCH * CP
CW = 104
KOFFS = (0, 16, 32, 48, 64, 80, 88)
VP = 2048
BINS_T = SENTINEL // 16
HIST_SZ = SENTINEL + 16
DUMP = MAXV
DUMP_PACK = DUMP * 32


def _ids_group(x, y, z, nvalid):
    lox, loy, loz = jnp.float32(-1.0), jnp.float32(-2.0), jnp.float32(-2.0)
    hix, hiy, hiz = jnp.float32(1.0), jnp.float32(2.0), jnp.float32(2.0)
    vs = jnp.float32(VSIZE)
    tx = ((x - lox) / vs).astype(jnp.int32)
    ty = ((y - loy) / vs).astype(jnp.int32)
    tz = ((z - loz) / vs).astype(jnp.int32)
    inr = ((x >= lox) & (x < hix) & (y >= loy) & (y < hiy)
           & (z >= loz) & (z < hiz)
           & (tx >= 0) & (tx < NX) & (ty >= 0) & (ty < NY)
           & (tz >= 0) & (tz < NZ))
    if nvalid < 16:
        inr = inr & (lax.iota(jnp.int32, 16) < nvalid)
    flat = tz * (NY * NX) + ty * NX + tx
    return jnp.where(inr, flat, SENTINEL)


def _sc_kernel(pcs_hbm, flat_hbm, enc_hbm, cc_hbm,
               hist_sh, acc_sh, cnt_sh, vid_sh, tot_sh,
               zbuf, zibuf, feat_v, hbuf, sbuf, ids_v, pbuf, sl2, cvals,
               cbuf, totb, tbuf, xidx, xb, yb, zb, cobuf, ofeat, sem):
    core = lax.axis_index("c")
    sub = lax.axis_index("s")
    lanes = lax.iota(jnp.int32, 16)

    def zloop(r, _):
        for off in KOFFS:
            zbuf[r, pl.ds(off, 16)] = jnp.zeros((16,), jnp.float32)
        return 0
    lax.fori_loop(0, CP, zloop, 0)

    def ziloop(i, _):
        zibuf[pl.ds(i * 16, 16)] = jnp.zeros((16,), jnp.int32)
        return 0
    lax.fori_loop(0, BINS_T // 16, ziloop, 0)

    def gather_ids(b, ck, i, nvalid):
        fbase = (b * N + ck * CP) * 3
        for g in range(CP // 16):
            p = g * 16 + lanes
            p = jnp.where(p < nvalid, p, 0)
            xidx[pl.ds(g * 16, 16)] = fbase + p * 3
        pltpu.async_copy(flat_hbm.at[xidx], xb, sem).wait()
        for g in range(CP // 16):
            xidx[pl.ds(g * 16, 16)] = xidx[pl.ds(g * 16, 16)] + 1
        pltpu.async_copy(flat_hbm.at[xidx], yb, sem).wait()
        for g in range(CP // 16):
            xidx[pl.ds(g * 16, 16)] = xidx[pl.ds(g * 16, 16)] + 1
        pltpu.async_copy(flat_hbm.at[xidx], zb, sem).wait()
        for g in range(CP // 16):
            nval = max(0, min(16, nvalid - g * 16))
            idv = _ids_group(xb[pl.ds(g * 16, 16)], yb[pl.ds(g * 16, 16)],
                             zb[pl.ds(g * 16, 16)], nval)
            ids_v[i, pl.ds(g * 16, 16)] = idv

    def one_batch(bb, _):
        b = 2 * core + bb

        pltpu.sync_copy(zbuf, acc_sh.at[pl.ds(sub * CP, CP), :])
        pltpu.sync_copy(zibuf.at[pl.ds(0, BINS_T)],
                        hist_sh.at[pl.ds(sub * BINS_T, BINS_T)])

        @pl.when(sub == 0)
        def _():
            pltpu.sync_copy(zibuf.at[pl.ds(0, VP)], cnt_sh)
            pltpu.sync_copy(zibuf.at[pl.ds(0, VP)], vid_sh)
            pltpu.sync_copy(zibuf.at[pl.ds(0, 16)],
                            hist_sh.at[pl.ds(SENTINEL, 16)])

        def oloop(i, _):
            cvals[pl.ds(i * 16, 16)] = jnp.full((16,), 1, jnp.int32)
            return 0
        lax.fori_loop(0, CP // 16, oloop, 0)
        plsc.subcore_barrier()

        def p1_chunk(i, _):
            ck = sub + 16 * i

            @pl.when(ck < NCH)
            def _():
                gather_ids(b, ck, i, CP)
                pltpu.sync_copy(cvals, hist_sh.at[ids_v.at[i]], add=True)
            return 0
        lax.fori_loop(0, CPT, p1_chunk, 0)

        @pl.when(sub == 0)
        def _():
            gather_ids(b, NCH, CPT, TAIL)
            pltpu.sync_copy(cvals, hist_sh.at[ids_v.at[CPT]], add=True)
        plsc.subcore_barrier()

        bbase = sub * BINS_T
        pltpu.sync_copy(hist_sh.at[pl.ds(bbase, BINS_T)], hbuf)

        def scan1(i, carry):
            v = hbuf[pl.ds(i * 16, 16)]
            occ = jnp.where(v > 0, 1, 0).astype(jnp.int32)
            x = occ
            for k in (1, 2, 4, 8):
                sh = x.at[jnp.maximum(lanes - k, 0)].get(
                    mode="promise_in_bounds")
                x = x + jnp.where(lanes >= k, sh, 0)
            inc = x + carry
            sbuf[pl.ds(i * 16, 16)] = inc - occ
            return inc[15]
        total = lax.fori_loop(0, BINS_T // 16, scan1, jnp.int32(0))

        totb[...] = jnp.full((16,), total, jnp.int32)
        pltpu.sync_copy(totb, tot_sh.at[pl.ds(sub * 16, 16)])
        plsc.subcore_barrier()
        pltpu.sync_copy(tot_sh, tbuf)
        base = jnp.int32(0)
        for t in range(16):
            tv = tbuf[pl.ds(t * 16, 16)]
            base = base + jnp.where(t < sub, tv[0], 0)

        def scan2(i, _):
            v = hbuf[pl.ds(i * 16, 16)]
            slot = sbuf[pl.ds(i * 16, 16)] + base
            valid = (v > 0) & (slot < MAXV)
            packed = jnp.where(valid, slot * 32 + jnp.minimum(v, MAXP),
                               DUMP_PACK)
            sbuf[pl.ds(i * 16, 16)] = packed
            return 0
        lax.fori_loop(0, BINS_T // 16, scan2, 0)
        pltpu.sync_copy(sbuf, hist_sh.at[pl.ds(bbase, BINS_T)])

        @pl.when(sub == 0)
        def _():
            totb[...] = jnp.full((16,), DUMP_PACK, jnp.int32)
            pltpu.sync_copy(totb, hist_sh.at[pl.ds(SENTINEL, 16)])
        plsc.subcore_barrier()

        def p3_work(i, ck, n):
            pltpu.async_copy(hist_sh.at[ids_v.at[i]], pbuf, sem).wait()
            for g in range(CP // 16):
                pv = pbuf[pl.ds(g * 16, 16)]
                sl2[0, pl.ds(g * 16, 16)] = pv >> 5
                cvals[pl.ds(g * 16, 16)] = pv & 31
            pltpu.sync_copy(pcs_hbm.at[b, pl.ds(ck * CP, n), :],
                            feat_v.at[pl.ds(0, n), :])
            pltpu.sync_copy(cvals, cnt_sh.at[sl2.at[0]])
            pltpu.sync_copy(ids_v.at[i], vid_sh.at[sl2.at[0]])
            pltpu.sync_copy(feat_v, acc_sh.at[sl2.at[0]], add=True)

        def p3_chunk(i, _):
            ck = sub + 16 * i

            @pl.when(ck < NCH)
            def _():
                p3_work(i, ck, CP)
            return 0
        lax.fori_loop(0, CPT, p3_chunk, 0)

        @pl.when(sub == 0)
        def _():
            p3_work(CPT, NCH, TAIL)
        plsc.subcore_barrier()

        r0 = sub * CP
        pltpu.sync_copy(acc_sh.at[pl.ds(r0, CP), :], feat_v)
        pltpu.sync_copy(cnt_sh.at[pl.ds(r0, CP)], cbuf)
        pltpu.sync_copy(vid_sh.at[pl.ds(r0, CP)], pbuf)

        def frow(g, _):
            cv = cbuf[pl.ds(g * 16, 16)].astype(jnp.float32)
            cf = jnp.maximum(cv, jnp.float32(1.0))
            scale16 = jnp.float32(MAXP) / (cf * cf)
            for rl in range(16):
                r = g * 16 + rl
                scale = scale16[rl]
                for off in KOFFS:
                    ofeat[r, pl.ds(off, 16)] = (
                        feat_v[r, pl.ds(off, 16)] * scale)
            return 0
        lax.fori_loop(0, CP // 16, frow, 0)
        pltpu.sync_copy(ofeat, enc_hbm.at[b, pl.ds(r0, CP), :])

        for g in range(CP // 16):
            vv = pbuf[pl.ds(g * 16, 16)]
            cnt = cbuf[pl.ds(g * 16, 16)]
            zc = (vv.astype(jnp.float32) / jnp.float32(NY * NX)).astype(
                jnp.int32)
            rem = vv - zc * (NY * NX)
            yc = (rem.astype(jnp.float32) / jnp.float32(NX)).astype(jnp.int32)
            xc = rem - yc * NX
            cobuf[pl.ds(g * 16, 16)] = zc
            cobuf[pl.ds(CP + g * 16, 16)] = yc
            cobuf[pl.ds(2 * CP + g * 16, 16)] = xc
            cobuf[pl.ds(3 * CP + g * 16, 16)] = cnt
        for p in range(4):
            pltpu.sync_copy(cobuf.at[pl.ds(p * CP, CP)],
                            cc_hbm.at[pl.ds((b * 4 + p) * VP + r0, CP)])
        plsc.subcore_barrier()
        return 0

    lax.fori_loop(0, B // 2, one_batch, 0)


@jax.jit
def kernel(point_cloud_features):
    mesh = plsc.VectorSubcoreMesh(core_axis_name="c", subcore_axis_name="s")
    run = functools.partial(
        pl.kernel,
        out_type=[
            jax.ShapeDtypeStruct((B, VP, CW), jnp.float32),
            jax.ShapeDtypeStruct((B * VP * 4,), jnp.int32),
        ],
        mesh=mesh,
        compiler_params=pltpu.CompilerParams(use_tc_tiling_on_sc=False),
        scratch_types=[
            pltpu.VMEM_SHARED((HIST_SZ,), jnp.int32),
            pltpu.VMEM_SHARED((VP, CW), jnp.float32),
            pltpu.VMEM_SHARED((VP,), jnp.int32),
            pltpu.VMEM_SHARED((VP,), jnp.int32),
            pltpu.VMEM_SHARED((256,), jnp.int32),
            pltpu.VMEM((CP, CW), jnp.float32),
            pltpu.VMEM((BINS_T,), jnp.int32),
            pltpu.VMEM((CP, CW), jnp.float32),
            pltpu.VMEM((BINS_T,), jnp.int32),
            pltpu.VMEM((BINS_T,), jnp.int32),
            pltpu.VMEM((CPT + 1, CP), jnp.int32),
            pltpu.VMEM((CP,), jnp.int32),
            pltpu.VMEM((1, CP), jnp.int32),
            pltpu.VMEM((CP,), jnp.int32),
            pltpu.VMEM((CP,), jnp.int32),
            pltpu.VMEM((16,), jnp.int32),
            pltpu.VMEM((256,), jnp.int32),
            pltpu.VMEM((CP,), jnp.int32),
            pltpu.VMEM((CP,), jnp.float32),
            pltpu.VMEM((CP,), jnp.float32),
            pltpu.VMEM((CP,), jnp.float32),
            pltpu.VMEM((CP * 4,), jnp.int32),
            pltpu.VMEM((CP, CW), jnp.float32),
            pltpu.SemaphoreType.DMA,
        ],
    )(_sc_kernel)
    pcs_p = jnp.pad(point_cloud_features, ((0, 0), (0, 0), (0, CW - C)))
    xyzf = point_cloud_features[:, :, :3].reshape(-1)
    enc_p, cc_p = run(pcs_p, xyzf)
    cc = cc_p.reshape(B, 4, VP)
    coords = jnp.stack([cc[:, 0, :MAXV], cc[:, 1, :MAXV], cc[:, 2, :MAXV]],
                       axis=2)
    return (enc_p[:, :MAXV, :C], coords, cc[:, 3, :MAXV])


# speedup vs baseline: 4.5456x; 1.0683x over previous
import functools

import jax
import jax.numpy as jnp
from jax import lax
from jax.experimental import pallas as pl
from jax.experimental.pallas import tpu as pltpu
from jax.experimental.pallas import tpu_sc as plsc

B = 4
N = 50000
C = 101
NX, NY, NZ = 40, 80, 80
VSIZE = 0.05
MAXV = 2000
MAXP = 25
SENTINEL = NX * NY * NZ

CP = 128
NCH = N // CP
CPT = (NCH + 15) // 16
TAIL = N - NCH * CP
CW = 104
KOFFS = (0, 16, 32, 48, 64, 80, 88)
VP = 2048
BINS_T = SENTINEL // 16
HIST_SZ = SENTINEL + 16
DUMP = MAXV
DUMP_PACK = DUMP * 32


def _ids_group(x, y, z, nvalid):
    lox, loy, loz = jnp.float32(-1.0), jnp.float32(-2.0), jnp.float32(-2.0)
    hix, hiy, hiz = jnp.float32(1.0), jnp.float32(2.0), jnp.float32(2.0)
    vs = jnp.float32(VSIZE)
    tx = ((x - lox) / vs).astype(jnp.int32)
    ty = ((y - loy) / vs).astype(jnp.int32)
    tz = ((z - loz) / vs).astype(jnp.int32)
    inr = ((x >= lox) & (x < hix) & (y >= loy) & (y < hiy)
           & (z >= loz) & (z < hiz)
           & (tx >= 0) & (tx < NX) & (ty >= 0) & (ty < NY)
           & (tz >= 0) & (tz < NZ))
    if nvalid < 16:
        inr = inr & (lax.iota(jnp.int32, 16) < nvalid)
    flat = tz * (NY * NX) + ty * NX + tx
    return jnp.where(inr, flat, SENTINEL)


def _sc_kernel(pcs_hbm, flat_hbm, enc_hbm, cc_hbm,
               hist_sh, acc_sh, cnt_sh, vid_sh, tot_sh,
               zbuf, zibuf, feat_v, hbuf, sbuf, ids_v, pbuf, sl2, cvals,
               cbuf, totb, tbuf, xidx, yidx, zidx, xb, yb, zb, cobuf, ofeat,
               sem, sem2, sem3):
    core = lax.axis_index("c")
    sub = lax.axis_index("s")
    lanes = lax.iota(jnp.int32, 16)

    def zloop(r, _):
        for off in KOFFS:
            zbuf[r, pl.ds(off, 16)] = jnp.zeros((16,), jnp.float32)
        return 0
    lax.fori_loop(0, CP, zloop, 0)

    def ziloop(i, _):
        zibuf[pl.ds(i * 16, 16)] = jnp.zeros((16,), jnp.int32)
        return 0
    lax.fori_loop(0, BINS_T // 16, ziloop, 0)

    def gather_ids(b, ck, i, nvalid):
        fbase = (b * N + ck * CP) * 3
        for g in range(CP // 16):
            p = g * 16 + lanes
            p = jnp.where(p < nvalid, p, 0)
            ix = fbase + p * 3
            xidx[pl.ds(g * 16, 16)] = ix
            yidx[pl.ds(g * 16, 16)] = ix + 1
            zidx[pl.ds(g * 16, 16)] = ix + 2
        cx = pltpu.async_copy(flat_hbm.at[xidx], xb, sem)
        cy = pltpu.async_copy(flat_hbm.at[yidx], yb, sem2)
        cz = pltpu.async_copy(flat_hbm.at[zidx], zb, sem3)
        cx.wait()
        cy.wait()
        cz.wait()
        for g in range(CP // 16):
            nval = max(0, min(16, nvalid - g * 16))
            idv = _ids_group(xb[pl.ds(g * 16, 16)], yb[pl.ds(g * 16, 16)],
                             zb[pl.ds(g * 16, 16)], nval)
            ids_v[i, pl.ds(g * 16, 16)] = idv

    def one_batch(bb, _):
        b = 2 * core + bb

        pltpu.sync_copy(zbuf, acc_sh.at[pl.ds(sub * CP, CP), :])
        pltpu.sync_copy(zibuf.at[pl.ds(0, BINS_T)],
                        hist_sh.at[pl.ds(sub * BINS_T, BINS_T)])

        @pl.when(sub == 0)
        def _():
            pltpu.sync_copy(zibuf.at[pl.ds(0, VP)], cnt_sh)
            pltpu.sync_copy(zibuf.at[pl.ds(0, VP)], vid_sh)
            pltpu.sync_copy(zibuf.at[pl.ds(0, 16)],
                            hist_sh.at[pl.ds(SENTINEL, 16)])

        def oloop(i, _):
            cvals[pl.ds(i * 16, 16)] = jnp.full((16,), 1, jnp.int32)
            return 0
        lax.fori_loop(0, CP // 16, oloop, 0)
        plsc.subcore_barrier()

        def p1_chunk(i, _):
            ck = sub + 16 * i

            @pl.when(ck < NCH)
            def _():
                gather_ids(b, ck, i, CP)
                pltpu.sync_copy(cvals, hist_sh.at[ids_v.at[i]], add=True)
            return 0
        lax.fori_loop(0, CPT, p1_chunk, 0)

        @pl.when(sub == 0)
        def _():
            gather_ids(b, NCH, CPT, TAIL)
            pltpu.sync_copy(cvals, hist_sh.at[ids_v.at[CPT]], add=True)
        plsc.subcore_barrier()

        bbase = sub * BINS_T
        pltpu.sync_copy(hist_sh.at[pl.ds(bbase, BINS_T)], hbuf)

        def pass_a(i, acc):
            v = hbuf[pl.ds(i * 16, 16)]
            occ = jnp.where(v > 0, 1, 0).astype(jnp.int32)
            sbuf[pl.ds(i * 16, 16)] = occ
            return acc + occ
        acc16 = lax.fori_loop(0, BINS_T // 16, pass_a,
                              jnp.zeros((16,), jnp.int32))
        total = jnp.int32(0)
        for l in range(16):
            total = total + acc16[l]

        totb[...] = jnp.full((16,), total, jnp.int32)
        pltpu.sync_copy(totb, tot_sh.at[pl.ds(sub * 16, 16)])
        plsc.subcore_barrier()
        pltpu.sync_copy(tot_sh, tbuf)
        base = jnp.int32(0)
        for t in range(16):
            tv = tbuf[pl.ds(t * 16, 16)]
            base = base + jnp.where(t < sub, tv[0], 0)

        @pl.when(base < MAXV)
        def _():
            def pass_b(i, carry):
                live = carry + base < MAXV

                def slow(c):
                    v = hbuf[pl.ds(i * 16, 16)]
                    occ = sbuf[pl.ds(i * 16, 16)]
                    x = occ
                    for k in (1, 2, 4, 8):
                        sh = x.at[jnp.maximum(lanes - k, 0)].get(
                            mode="promise_in_bounds")
                        x = x + jnp.where(lanes >= k, sh, 0)
                    inc = x + c
                    slot = inc - occ + base
                    valid = (occ > 0) & (slot < MAXV)
                    packed = jnp.where(valid,
                                       slot * 32 + jnp.minimum(v, MAXP),
                                       DUMP_PACK)
                    sbuf[pl.ds(i * 16, 16)] = packed
                    return inc[15]

                def fast(c):
                    sbuf[pl.ds(i * 16, 16)] = jnp.full((16,), DUMP_PACK,
                                                       jnp.int32)
                    return c
                return lax.cond(live, slow, fast, carry)
            lax.fori_loop(0, BINS_T // 16, pass_b, jnp.int32(0))

        @pl.when(base >= MAXV)
        def _():
            def dump_fill(i, _):
                sbuf[pl.ds(i * 16, 16)] = jnp.full((16,), DUMP_PACK,
                                                   jnp.int32)
                return 0
            lax.fori_loop(0, BINS_T // 16, dump_fill, 0)
        pltpu.sync_copy(sbuf, hist_sh.at[pl.ds(bbase, BINS_T)])

        @pl.when(sub == 0)
        def _():
            totb[...] = jnp.full((16,), DUMP_PACK, jnp.int32)
            pltpu.sync_copy(totb, hist_sh.at[pl.ds(SENTINEL, 16)])
        plsc.subcore_barrier()

        def p3_work(i, ck, n):
            cg = pltpu.async_copy(hist_sh.at[ids_v.at[i]], pbuf, sem)
            cf = pltpu.async_copy(pcs_hbm.at[b, pl.ds(ck * CP, n), :],
                                  feat_v.at[pl.ds(0, n), :], sem2)
            cg.wait()
            for g in range(CP // 16):
                pv = pbuf[pl.ds(g * 16, 16)]
                sl2[0, pl.ds(g * 16, 16)] = pv >> 5
                cvals[pl.ds(g * 16, 16)] = pv & 31
            cf.wait()
            pltpu.sync_copy(cvals, cnt_sh.at[sl2.at[0]])
            pltpu.sync_copy(ids_v.at[i], vid_sh.at[sl2.at[0]])
            pltpu.sync_copy(feat_v, acc_sh.at[sl2.at[0]], add=True)

        def p3_chunk(i, _):
            ck = sub + 16 * i

            @pl.when(ck < NCH)
            def _():
                p3_work(i, ck, CP)
            return 0
        lax.fori_loop(0, CPT, p3_chunk, 0)

        @pl.when(sub == 0)
        def _():
            p3_work(CPT, NCH, TAIL)
        plsc.subcore_barrier()

        r0 = sub * CP
        pltpu.sync_copy(acc_sh.at[pl.ds(r0, CP), :], feat_v)
        pltpu.sync_copy(cnt_sh.at[pl.ds(r0, CP)], cbuf)
        pltpu.sync_copy(vid_sh.at[pl.ds(r0, CP)], pbuf)

        def frow(g, _):
            cv = cbuf[pl.ds(g * 16, 16)].astype(jnp.float32)
            cf = jnp.maximum(cv, jnp.float32(1.0))
            scale16 = jnp.float32(MAXP) / (cf * cf)
            for rl in range(16):
                r = g * 16 + rl
                scale = scale16[rl]
                for off in KOFFS:
                    ofeat[r, pl.ds(off, 16)] = (
                        feat_v[r, pl.ds(off, 16)] * scale)
            return 0
        lax.fori_loop(0, CP // 16, frow, 0)
        pltpu.sync_copy(ofeat, enc_hbm.at[b, pl.ds(r0, CP), :])

        for g in range(CP // 16):
            vv = pbuf[pl.ds(g * 16, 16)]
            cnt = cbuf[pl.ds(g * 16, 16)]
            zc = (vv.astype(jnp.float32) / jnp.float32(NY * NX)).astype(
                jnp.int32)
            rem = vv - zc * (NY * NX)
            yc = (rem.astype(jnp.float32) / jnp.float32(NX)).astype(jnp.int32)
            xc = rem - yc * NX
            cobuf[pl.ds(g * 16, 16)] = zc
            cobuf[pl.ds(CP + g * 16, 16)] = yc
            cobuf[pl.ds(2 * CP + g * 16, 16)] = xc
            cobuf[pl.ds(3 * CP + g * 16, 16)] = cnt
        for p in range(4):
            pltpu.sync_copy(cobuf.at[pl.ds(p * CP, CP)],
                            cc_hbm.at[pl.ds((b * 4 + p) * VP + r0, CP)])
        plsc.subcore_barrier()
        return 0

    lax.fori_loop(0, B // 2, one_batch, 0)


@jax.jit
def kernel(point_cloud_features):
    mesh = plsc.VectorSubcoreMesh(core_axis_name="c", subcore_axis_name="s")
    run = functools.partial(
        pl.kernel,
        out_type=[
            jax.ShapeDtypeStruct((B, VP, CW), jnp.float32),
            jax.ShapeDtypeStruct((B * VP * 4,), jnp.int32),
        ],
        mesh=mesh,
        compiler_params=pltpu.CompilerParams(use_tc_tiling_on_sc=False),
        scratch_types=[
            pltpu.VMEM_SHARED((HIST_SZ,), jnp.int32),
            pltpu.VMEM_SHARED((VP, CW), jnp.float32),
            pltpu.VMEM_SHARED((VP,), jnp.int32),
            pltpu.VMEM_SHARED((VP,), jnp.int32),
            pltpu.VMEM_SHARED((256,), jnp.int32),
            pltpu.VMEM((CP, CW), jnp.float32),
            pltpu.VMEM((BINS_T,), jnp.int32),
            pltpu.VMEM((CP, CW), jnp.float32),
            pltpu.VMEM((BINS_T,), jnp.int32),
            pltpu.VMEM((BINS_T,), jnp.int32),
            pltpu.VMEM((CPT + 1, CP), jnp.int32),
            pltpu.VMEM((CP,), jnp.int32),
            pltpu.VMEM((1, CP), jnp.int32),
            pltpu.VMEM((CP,), jnp.int32),
            pltpu.VMEM((CP,), jnp.int32),
            pltpu.VMEM((16,), jnp.int32),
            pltpu.VMEM((256,), jnp.int32),
            pltpu.VMEM((CP,), jnp.int32),
            pltpu.VMEM((CP,), jnp.int32),
            pltpu.VMEM((CP,), jnp.int32),
            pltpu.VMEM((CP,), jnp.float32),
            pltpu.VMEM((CP,), jnp.float32),
            pltpu.VMEM((CP,), jnp.float32),
            pltpu.VMEM((CP * 4,), jnp.int32),
            pltpu.VMEM((CP, CW), jnp.float32),
            pltpu.SemaphoreType.DMA,
            pltpu.SemaphoreType.DMA,
            pltpu.SemaphoreType.DMA,
        ],
    )(_sc_kernel)
    pcs_p = jnp.pad(point_cloud_features, ((0, 0), (0, 0), (0, CW - C)))
    xyzf = point_cloud_features[:, :, :3].reshape(-1)
    enc_p, cc_p = run(pcs_p, xyzf)
    cc = cc_p.reshape(B, 4, VP)
    coords = jnp.stack([cc[:, 0, :MAXV], cc[:, 1, :MAXV], cc[:, 2, :MAXV]],
                       axis=2)
    return (enc_p[:, :MAXV, :C], coords, cc[:, 3, :MAXV])


# speedup vs baseline: 4.6618x; 1.0256x over previous
import functools

import jax
import jax.numpy as jnp
from jax import lax
from jax.experimental import pallas as pl
from jax.experimental.pallas import tpu as pltpu
from jax.experimental.pallas import tpu_sc as plsc

B = 4
N = 50000
C = 101
NX, NY, NZ = 40, 80, 80
VSIZE = 0.05
MAXV = 2000
MAXP = 25
SENTINEL = NX * NY * NZ

CP = 128
NCH = N // CP
CPT = (NCH + 15) // 16
TAIL = N - NCH * CP
CW = 104
KOFFS = (0, 16, 32, 48, 64, 80, 88)
VP = 2048
BINS_T = SENTINEL // 16
HIST_SZ = SENTINEL + 16
DUMP = MAXV
DUMP_PACK = DUMP * 32


def _ids_group(x, y, z, nvalid):
    lox, loy, loz = jnp.float32(-1.0), jnp.float32(-2.0), jnp.float32(-2.0)
    hix, hiy, hiz = jnp.float32(1.0), jnp.float32(2.0), jnp.float32(2.0)
    vs = jnp.float32(VSIZE)
    tx = ((x - lox) / vs).astype(jnp.int32)
    ty = ((y - loy) / vs).astype(jnp.int32)
    tz = ((z - loz) / vs).astype(jnp.int32)
    inr = ((x >= lox) & (x < hix) & (y >= loy) & (y < hiy)
           & (z >= loz) & (z < hiz)
           & (tx >= 0) & (tx < NX) & (ty >= 0) & (ty < NY)
           & (tz >= 0) & (tz < NZ))
    if nvalid < 16:
        inr = inr & (lax.iota(jnp.int32, 16) < nvalid)
    flat = tz * (NY * NX) + ty * NX + tx
    return jnp.where(inr, flat, SENTINEL)


def _sc_kernel(pcs_hbm, flat_hbm, enc_hbm, cc_hbm,
               hist_sh, acc_sh, cv_sh, tot_sh,
               zbuf, zibuf, fv2, hbuf, sbuf, ids_v, pb2, sl2, cvals,
               totb, tbuf, xidx, yidx, zidx, xb2, yb2, zb2, cobuf, ofeat,
               sem, gsx, gsy, gsz, psm, fsm):
    core = lax.axis_index("c")
    sub = lax.axis_index("s")
    lanes = lax.iota(jnp.int32, 16)

    def zloop(r, _):
        for off in KOFFS:
            zbuf[r, pl.ds(off, 16)] = jnp.zeros((16,), jnp.float32)
        return 0
    lax.fori_loop(0, CP, zloop, 0)

    def ziloop(i, _):
        zibuf[pl.ds(i * 16, 16)] = jnp.zeros((16,), jnp.int32)
        return 0
    lax.fori_loop(0, VP // 16, ziloop, 0)

    def one_batch(bb, _):
        b = 2 * core + bb

        pltpu.sync_copy(zbuf, acc_sh.at[pl.ds(sub * CP, CP), :])
        for q in range(BINS_T // VP):
            pltpu.sync_copy(zibuf,
                            hist_sh.at[pl.ds(sub * BINS_T + q * VP, VP)])
        rem0 = BINS_T - (BINS_T // VP) * VP
        if rem0:
            pltpu.sync_copy(
                zibuf.at[pl.ds(0, rem0)],
                hist_sh.at[pl.ds(sub * BINS_T + BINS_T - rem0, rem0)])

        @pl.when(sub == 0)
        def _():
            pltpu.sync_copy(zibuf, cv_sh)
            pltpu.sync_copy(zibuf.at[pl.ds(0, 16)],
                            hist_sh.at[pl.ds(SENTINEL, 16)])

        def oloop(i, _):
            cvals[pl.ds(i * 16, 16)] = jnp.full((16,), 1, jnp.int32)
            return 0
        lax.fori_loop(0, CP // 16, oloop, 0)
        plsc.subcore_barrier()

        def p1_issue(i, par):
            ck = sub + 16 * i

            @pl.when(ck < NCH)
            def _():
                fbase = (b * N + ck * CP) * 3
                for g in range(CP // 16):
                    ix = fbase + (g * 16 + lanes) * 3
                    xidx[par, pl.ds(g * 16, 16)] = ix
                    yidx[par, pl.ds(g * 16, 16)] = ix + 1
                    zidx[par, pl.ds(g * 16, 16)] = ix + 2
                pltpu.async_copy(flat_hbm.at[xidx.at[par]], xb2.at[par],
                                 gsx.at[par])
                pltpu.async_copy(flat_hbm.at[yidx.at[par]], yb2.at[par],
                                 gsy.at[par])
                pltpu.async_copy(flat_hbm.at[zidx.at[par]], zb2.at[par],
                                 gsz.at[par])

        p1_issue(0, 0)

        def p1_chunk(i, _):
            par = i & 1
            ck = sub + 16 * i

            @pl.when(ck < NCH)
            def _():
                pltpu.make_async_copy(flat_hbm.at[xidx.at[par]],
                                      xb2.at[par], gsx.at[par]).wait()
                pltpu.make_async_copy(flat_hbm.at[yidx.at[par]],
                                      yb2.at[par], gsy.at[par]).wait()
                pltpu.make_async_copy(flat_hbm.at[zidx.at[par]],
                                      zb2.at[par], gsz.at[par]).wait()
                for g in range(CP // 16):
                    idv = _ids_group(xb2[par, pl.ds(g * 16, 16)],
                                     yb2[par, pl.ds(g * 16, 16)],
                                     zb2[par, pl.ds(g * 16, 16)], 16)
                    ids_v[i, pl.ds(g * 16, 16)] = idv
            p1_issue(i + 1, 1 - par)

            @pl.when(ck < NCH)
            def _():
                pltpu.sync_copy(cvals, hist_sh.at[ids_v.at[i]], add=True)
            return 0
        lax.fori_loop(0, CPT, p1_chunk, 0)

        @pl.when(sub == 0)
        def _():
            fbase = (b * N + NCH * CP) * 3
            for g in range(CP // 16):
                p = g * 16 + lanes
                p = jnp.where(p < TAIL, p, 0)
                ix = fbase + p * 3
                xidx[0, pl.ds(g * 16, 16)] = ix
                yidx[0, pl.ds(g * 16, 16)] = ix + 1
                zidx[0, pl.ds(g * 16, 16)] = ix + 2
            cx = pltpu.async_copy(flat_hbm.at[xidx.at[0]], xb2.at[0],
                                  gsx.at[0])
            cy = pltpu.async_copy(flat_hbm.at[yidx.at[0]], yb2.at[0],
                                  gsy.at[0])
            cz = pltpu.async_copy(flat_hbm.at[zidx.at[0]], zb2.at[0],
                                  gsz.at[0])
            cx.wait()
            cy.wait()
            cz.wait()
            for g in range(CP // 16):
                nval = max(0, min(16, TAIL - g * 16))
                idv = _ids_group(xb2[0, pl.ds(g * 16, 16)],
                                 yb2[0, pl.ds(g * 16, 16)],
                                 zb2[0, pl.ds(g * 16, 16)], nval)
                ids_v[CPT, pl.ds(g * 16, 16)] = idv
            pltpu.sync_copy(cvals, hist_sh.at[ids_v.at[CPT]], add=True)
        plsc.subcore_barrier()

        bbase = sub * BINS_T
        pltpu.sync_copy(hist_sh.at[pl.ds(bbase, BINS_T)], hbuf)

        def pass_a(i, acc):
            v = hbuf[pl.ds(i * 16, 16)]
            occ = jnp.where(v > 0, 1, 0).astype(jnp.int32)
            sbuf[pl.ds(i * 16, 16)] = occ
            return acc + occ
        acc16 = lax.fori_loop(0, BINS_T // 16, pass_a,
                              jnp.zeros((16,), jnp.int32))
        total = jnp.int32(0)
        for l in range(16):
            total = total + acc16[l]

        totb[...] = jnp.full((16,), total, jnp.int32)
        pltpu.sync_copy(totb, tot_sh.at[pl.ds(sub * 16, 16)])
        plsc.subcore_barrier()
        pltpu.sync_copy(tot_sh, tbuf)
        base = jnp.int32(0)
        for t in range(16):
            tv = tbuf[pl.ds(t * 16, 16)]
            base = base + jnp.where(t < sub, tv[0], 0)

        @pl.when(base < MAXV)
        def _():
            def pass_b(i, carry):
                live = carry + base < MAXV

                def slow(c):
                    v = hbuf[pl.ds(i * 16, 16)]
                    occ = sbuf[pl.ds(i * 16, 16)]
                    x = occ
                    for k in (1, 2, 4, 8):
                        sh = x.at[jnp.maximum(lanes - k, 0)].get(
                            mode="promise_in_bounds")
                        x = x + jnp.where(lanes >= k, sh, 0)
                    inc = x + c
                    slot = inc - occ + base
                    valid = (occ > 0) & (slot < MAXV)
                    packed = jnp.where(valid,
                                       slot * 32 + jnp.minimum(v, MAXP),
                                       DUMP_PACK)
                    sbuf[pl.ds(i * 16, 16)] = packed
                    return inc[15]

                def fast(c):
                    sbuf[pl.ds(i * 16, 16)] = jnp.full((16,), DUMP_PACK,
                                                       jnp.int32)
                    return c
                return lax.cond(live, slow, fast, carry)
            lax.fori_loop(0, BINS_T // 16, pass_b, jnp.int32(0))

        @pl.when(base >= MAXV)
        def _():
            def dump_fill(i, _):
                sbuf[pl.ds(i * 16, 16)] = jnp.full((16,), DUMP_PACK,
                                                   jnp.int32)
                return 0
            lax.fori_loop(0, BINS_T // 16, dump_fill, 0)
        pltpu.sync_copy(sbuf, hist_sh.at[pl.ds(bbase, BINS_T)])

        @pl.when(sub == 0)
        def _():
            totb[...] = jnp.full((16,), DUMP_PACK, jnp.int32)
            pltpu.sync_copy(totb, hist_sh.at[pl.ds(SENTINEL, 16)])
        plsc.subcore_barrier()

        def p3_issue(i, par):
            ck = sub + 16 * i

            @pl.when(ck < NCH)
            def _():
                pltpu.async_copy(hist_sh.at[ids_v.at[i]], pb2.at[par],
                                 psm.at[par])
                pltpu.async_copy(pcs_hbm.at[b, pl.ds(ck * CP, CP), :],
                                 fv2.at[par], fsm.at[par])

        p3_issue(0, 0)

        def p3_chunk(i, _):
            par = i & 1
            ck = sub + 16 * i

            @pl.when(ck < NCH)
            def _():
                pltpu.make_async_copy(hist_sh.at[ids_v.at[i]], pb2.at[par],
                                      psm.at[par]).wait()
                for g in range(CP // 16):
                    pv = pb2[par, pl.ds(g * 16, 16)]
                    sl2[0, pl.ds(g * 16, 16)] = pv >> 5
                    cvals[pl.ds(g * 16, 16)] = (
                        (ids_v[i, pl.ds(g * 16, 16)] << 5) | (pv & 31))
            p3_issue(i + 1, 1 - par)

            @pl.when(ck < NCH)
            def _():
                pltpu.make_async_copy(pcs_hbm.at[b, pl.ds(ck * CP, CP), :],
                                      fv2.at[par], fsm.at[par]).wait()
                pltpu.sync_copy(cvals, cv_sh.at[sl2.at[0]])
                pltpu.sync_copy(fv2.at[par], acc_sh.at[sl2.at[0]], add=True)
            return 0
        lax.fori_loop(0, CPT, p3_chunk, 0)

        @pl.when(sub == 0)
        def _():
            cg = pltpu.async_copy(hist_sh.at[ids_v.at[CPT]], pb2.at[0],
                                  psm.at[0])
            cf = pltpu.async_copy(pcs_hbm.at[b, pl.ds(NCH * CP, TAIL), :],
                                  fv2.at[0, pl.ds(0, TAIL), :],
                                  fsm.at[0])
            cg.wait()
            for g in range(CP // 16):
                pv = pb2[0, pl.ds(g * 16, 16)]
                sl2[0, pl.ds(g * 16, 16)] = pv >> 5
                cvals[pl.ds(g * 16, 16)] = (
                    (ids_v[CPT, pl.ds(g * 16, 16)] << 5) | (pv & 31))
            cf.wait()
            pltpu.sync_copy(cvals, cv_sh.at[sl2.at[0]])
            pltpu.sync_copy(fv2.at[0], acc_sh.at[sl2.at[0]], add=True)
        plsc.subcore_barrier()

        r0 = sub * CP
        pltpu.sync_copy(acc_sh.at[pl.ds(r0, CP), :], fv2.at[0])
        pltpu.sync_copy(cv_sh.at[pl.ds(r0, CP)], cvals)

        def frow(g, _):
            cv = (cvals[pl.ds(g * 16, 16)] & 31).astype(jnp.float32)
            cf = jnp.maximum(cv, jnp.float32(1.0))
            scale16 = jnp.float32(MAXP) / (cf * cf)
            for rl in range(16):
                r = g * 16 + rl
                scale = scale16[rl]
                for off in KOFFS:
                    ofeat[r, pl.ds(off, 16)] = (
                        fv2[0, r, pl.ds(off, 16)] * scale)
            return 0
        lax.fori_loop(0, CP // 16, frow, 0)
        pltpu.sync_copy(ofeat, enc_hbm.at[b, pl.ds(r0, CP), :])

        for g in range(CP // 16):
            cvv = cvals[pl.ds(g * 16, 16)]
            vv = cvv >> 5
            cnt = cvv & 31
            zc = (vv.astype(jnp.float32) / jnp.float32(NY * NX)).astype(
                jnp.int32)
            rem = vv - zc * (NY * NX)
            yc = (rem.astype(jnp.float32) / jnp.float32(NX)).astype(jnp.int32)
            xc = rem - yc * NX
            cobuf[pl.ds(g * 16, 16)] = zc
            cobuf[pl.ds(CP + g * 16, 16)] = yc
            cobuf[pl.ds(2 * CP + g * 16, 16)] = xc
            cobuf[pl.ds(3 * CP + g * 16, 16)] = cnt
        for p in range(4):
            pltpu.sync_copy(cobuf.at[pl.ds(p * CP, CP)],
                            cc_hbm.at[pl.ds((b * 4 + p) * VP + r0, CP)])
        plsc.subcore_barrier()
        return 0

    lax.fori_loop(0, B // 2, one_batch, 0)


@jax.jit
def kernel(point_cloud_features):
    mesh = plsc.VectorSubcoreMesh(core_axis_name="c", subcore_axis_name="s")
    run = functools.partial(
        pl.kernel,
        out_type=[
            jax.ShapeDtypeStruct((B, VP, CW), jnp.float32),
            jax.ShapeDtypeStruct((B * VP * 4,), jnp.int32),
        ],
        mesh=mesh,
        compiler_params=pltpu.CompilerParams(use_tc_tiling_on_sc=False),
        scratch_types=[
            pltpu.VMEM_SHARED((HIST_SZ,), jnp.int32),
            pltpu.VMEM_SHARED((VP, CW), jnp.float32),
            pltpu.VMEM_SHARED((VP,), jnp.int32),
            pltpu.VMEM_SHARED((256,), jnp.int32),
            pltpu.VMEM((CP, CW), jnp.float32),
            pltpu.VMEM((VP,), jnp.int32),
            pltpu.VMEM((2, CP, CW), jnp.float32),
            pltpu.VMEM((BINS_T,), jnp.int32),
            pltpu.VMEM((BINS_T,), jnp.int32),
            pltpu.VMEM((CPT + 1, CP), jnp.int32),
            pltpu.VMEM((2, CP), jnp.int32),
            pltpu.VMEM((1, CP), jnp.int32),
            pltpu.VMEM((CP,), jnp.int32),
            pltpu.VMEM((16,), jnp.int32),
            pltpu.VMEM((256,), jnp.int32),
            pltpu.VMEM((2, CP), jnp.int32),
            pltpu.VMEM((2, CP), jnp.int32),
            pltpu.VMEM((2, CP), jnp.int32),
            pltpu.VMEM((2, CP), jnp.float32),
            pltpu.VMEM((2, CP), jnp.float32),
            pltpu.VMEM((2, CP), jnp.float32),
            pltpu.VMEM((CP * 4,), jnp.int32),
            pltpu.VMEM((CP, CW), jnp.float32),
            pltpu.SemaphoreType.DMA,
            pltpu.SemaphoreType.DMA((2,)),
            pltpu.SemaphoreType.DMA((2,)),
            pltpu.SemaphoreType.DMA((2,)),
            pltpu.SemaphoreType.DMA((2,)),
            pltpu.SemaphoreType.DMA((2,)),
        ],
    )(_sc_kernel)
    pcs_p = jnp.pad(point_cloud_features, ((0, 0), (0, 0), (0, CW - C)))
    xyzf = point_cloud_features[:, :, :3].reshape(-1)
    enc_p, cc_p = run(pcs_p, xyzf)
    cc = cc_p.reshape(B, 4, VP)
    coords = jnp.stack([cc[:, 0, :MAXV], cc[:, 1, :MAXV], cc[:, 2, :MAXV]],
                       axis=2)
    return (enc_p[:, :MAXV, :C], coords, cc[:, 3, :MAXV])


# speedup vs baseline: 4.9744x; 1.0671x over previous
import functools

import jax
import jax.numpy as jnp
from jax import lax
from jax.experimental import pallas as pl
from jax.experimental.pallas import tpu as pltpu
from jax.experimental.pallas import tpu_sc as plsc

B = 4
N = 50000
C = 101
NX, NY, NZ = 40, 80, 80
VSIZE = 0.05
MAXV = 2000
MAXP = 25
SENTINEL = NX * NY * NZ

CP = 128
NCH = N // CP
CPT = (NCH + 15) // 16
TAIL = N - NCH * CP
CW = 104
KOFFS = (0, 16, 32, 48, 64, 80, 88)
VP = 2048
BINS_T = SENTINEL // 16
HIST_SZ = SENTINEL + 16
DUMP = MAXV
DUMP_PACK = DUMP * 32
REG = (CPT + 1) * CP
LIST_SZ = 16 * REG + 16
DUMPCELL = 16 * REG


def _ids_group(x, y, z, nvalid):
    lox, loy, loz = jnp.float32(-1.0), jnp.float32(-2.0), jnp.float32(-2.0)
    hix, hiy, hiz = jnp.float32(1.0), jnp.float32(2.0), jnp.float32(2.0)
    vs = jnp.float32(VSIZE)
    tx = ((x - lox) / vs).astype(jnp.int32)
    ty = ((y - loy) / vs).astype(jnp.int32)
    tz = ((z - loz) / vs).astype(jnp.int32)
    inr = ((x >= lox) & (x < hix) & (y >= loy) & (y < hiy)
           & (z >= loz) & (z < hiz)
           & (tx >= 0) & (tx < NX) & (ty >= 0) & (ty < NY)
           & (tz >= 0) & (tz < NZ))
    if nvalid < 16:
        inr = inr & (lax.iota(jnp.int32, 16) < nvalid)
    flat = tz * (NY * NX) + ty * NX + tx
    return jnp.where(inr, flat, SENTINEL)


def _sc_kernel(rows_hbm, flat_hbm, enc_hbm, cc_hbm,
               hist_sh, acc_sh, cv_sh, tot_sh, list_sh,
               zbuf, zibuf, fv1, hbuf, sbuf, ids_v, pb2, sl2, cvals,
               totb, tbuf, xidx, yidx, zidx, xb2, yb2, zb2, cobuf, ofeat,
               padc, posb, lbuf, gb, sidx, rv,
               sem, gsx, gsy, gsz, psm, fsm):
    core = lax.axis_index("c")
    sub = lax.axis_index("s")
    lanes = lax.iota(jnp.int32, 16)

    def zloop(r, _):
        for off in KOFFS:
            zbuf[r, pl.ds(off, 16)] = jnp.zeros((16,), jnp.float32)
        return 0
    lax.fori_loop(0, CP, zloop, 0)

    def ziloop(i, _):
        zibuf[pl.ds(i * 16, 16)] = jnp.zeros((16,), jnp.int32)
        return 0
    lax.fori_loop(0, VP // 16, ziloop, 0)

    def one_batch(bb, _):
        b = 2 * core + bb

        pltpu.sync_copy(zbuf, acc_sh.at[pl.ds(sub * CP, CP), :])
        for q in range(BINS_T // VP):
            pltpu.sync_copy(zibuf,
                            hist_sh.at[pl.ds(sub * BINS_T + q * VP, VP)])
        rem0 = BINS_T - (BINS_T // VP) * VP
        if rem0:
            pltpu.sync_copy(
                zibuf.at[pl.ds(0, rem0)],
                hist_sh.at[pl.ds(sub * BINS_T + BINS_T - rem0, rem0)])

        @pl.when(sub == 0)
        def _():
            pltpu.sync_copy(zibuf, cv_sh)
            pltpu.sync_copy(zibuf.at[pl.ds(0, 16)],
                            hist_sh.at[pl.ds(SENTINEL, 16)])

        def oloop(i, _):
            cvals[pl.ds(i * 16, 16)] = jnp.full((16,), 1, jnp.int32)
            return 0
        lax.fori_loop(0, CP // 16, oloop, 0)

        padval = (b * N) * 2048 + DUMP

        def ploop(i, _):
            padc[pl.ds(i * 16, 16)] = jnp.full((16,), padval, jnp.int32)
            return 0
        lax.fori_loop(0, VP // 16, ploop, 0)
        pltpu.sync_copy(padc, list_sh.at[pl.ds(sub * REG, VP)])
        pltpu.sync_copy(padc.at[pl.ds(0, REG - VP)],
                        list_sh.at[pl.ds(sub * REG + VP, REG - VP)])
        plsc.subcore_barrier()

        def p1_issue(i, par):
            ck = sub + 16 * i

            @pl.when(ck < NCH)
            def _():
                fbase = (b * N + ck * CP) * 3
                for g in range(CP // 16):
                    ix = fbase + (g * 16 + lanes) * 3
                    xidx[par, pl.ds(g * 16, 16)] = ix
                    yidx[par, pl.ds(g * 16, 16)] = ix + 1
                    zidx[par, pl.ds(g * 16, 16)] = ix + 2
                pltpu.async_copy(flat_hbm.at[xidx.at[par]], xb2.at[par],
                                 gsx.at[par])
                pltpu.async_copy(flat_hbm.at[yidx.at[par]], yb2.at[par],
                                 gsy.at[par])
                pltpu.async_copy(flat_hbm.at[zidx.at[par]], zb2.at[par],
                                 gsz.at[par])

        p1_issue(0, 0)

        def p1_chunk(i, _):
            par = i & 1
            ck = sub + 16 * i

            @pl.when(ck < NCH)
            def _():
                pltpu.make_async_copy(flat_hbm.at[xidx.at[par]],
                                      xb2.at[par], gsx.at[par]).wait()
                pltpu.make_async_copy(flat_hbm.at[yidx.at[par]],
                                      yb2.at[par], gsy.at[par]).wait()
                pltpu.make_async_copy(flat_hbm.at[zidx.at[par]],
                                      zb2.at[par], gsz.at[par]).wait()
                for g in range(CP // 16):
                    idv = _ids_group(xb2[par, pl.ds(g * 16, 16)],
                                     yb2[par, pl.ds(g * 16, 16)],
                                     zb2[par, pl.ds(g * 16, 16)], 16)
                    ids_v[i, pl.ds(g * 16, 16)] = idv
            p1_issue(i + 1, 1 - par)

            @pl.when(ck < NCH)
            def _():
                pltpu.sync_copy(cvals, hist_sh.at[ids_v.at[i]], add=True)
            return 0
        lax.fori_loop(0, CPT, p1_chunk, 0)

        @pl.when(sub == 0)
        def _():
            fbase = (b * N + NCH * CP) * 3
            for g in range(CP // 16):
                p = g * 16 + lanes
                p = jnp.where(p < TAIL, p, 0)
                ix = fbase + p * 3
                xidx[0, pl.ds(g * 16, 16)] = ix
                yidx[0, pl.ds(g * 16, 16)] = ix + 1
                zidx[0, pl.ds(g * 16, 16)] = ix + 2
            cx = pltpu.async_copy(flat_hbm.at[xidx.at[0]], xb2.at[0],
                                  gsx.at[0])
            cy = pltpu.async_copy(flat_hbm.at[yidx.at[0]], yb2.at[0],
                                  gsy.at[0])
            cz = pltpu.async_copy(flat_hbm.at[zidx.at[0]], zb2.at[0],
                                  gsz.at[0])
            cx.wait()
            cy.wait()
            cz.wait()
            for g in range(CP // 16):
                nval = max(0, min(16, TAIL - g * 16))
                idv = _ids_group(xb2[0, pl.ds(g * 16, 16)],
                                 yb2[0, pl.ds(g * 16, 16)],
                                 zb2[0, pl.ds(g * 16, 16)], nval)
                ids_v[CPT, pl.ds(g * 16, 16)] = idv
            pltpu.sync_copy(cvals, hist_sh.at[ids_v.at[CPT]], add=True)
        plsc.subcore_barrier()

        bbase = sub * BINS_T
        pltpu.sync_copy(hist_sh.at[pl.ds(bbase, BINS_T)], hbuf)

        def pass_a(i, acc):
            v = hbuf[pl.ds(i * 16, 16)]
            occ = jnp.where(v > 0, 1, 0).astype(jnp.int32)
            sbuf[pl.ds(i * 16, 16)] = occ
            return acc + occ
        acc16 = lax.fori_loop(0, BINS_T // 16, pass_a,
                              jnp.zeros((16,), jnp.int32))
        total = jnp.int32(0)
        for l in range(16):
            total = total + acc16[l]

        totb[...] = jnp.full((16,), total, jnp.int32)
        pltpu.sync_copy(totb, tot_sh.at[pl.ds(sub * 16, 16)])
        plsc.subcore_barrier()
        pltpu.sync_copy(tot_sh, tbuf)
        base = jnp.int32(0)
        for t in range(16):
            tv = tbuf[pl.ds(t * 16, 16)]
            base = base + jnp.where(t < sub, tv[0], 0)

        @pl.when(base < MAXV)
        def _():
            def pass_b(i, carry):
                live = carry + base < MAXV

                def slow(c):
                    v = hbuf[pl.ds(i * 16, 16)]
                    occ = sbuf[pl.ds(i * 16, 16)]
                    x = occ
                    for k in (1, 2, 4, 8):
                        sh = x.at[jnp.maximum(lanes - k, 0)].get(
                            mode="promise_in_bounds")
                        x = x + jnp.where(lanes >= k, sh, 0)
                    inc = x + c
                    slot = inc - occ + base
                    valid = (occ > 0) & (slot < MAXV)
                    packed = jnp.where(valid,
                                       slot * 32 + jnp.minimum(v, MAXP),
                                       DUMP_PACK)
                    sbuf[pl.ds(i * 16, 16)] = packed
                    return inc[15]

                def fast(c):
                    sbuf[pl.ds(i * 16, 16)] = jnp.full((16,), DUMP_PACK,
                                                       jnp.int32)
                    return c
                return lax.cond(live, slow, fast, carry)
            lax.fori_loop(0, BINS_T // 16, pass_b, jnp.int32(0))

        @pl.when(base >= MAXV)
        def _():
            def dump_fill(i, _):
                sbuf[pl.ds(i * 16, 16)] = jnp.full((16,), DUMP_PACK,
                                                   jnp.int32)
                return 0
            lax.fori_loop(0, BINS_T // 16, dump_fill, 0)
        pltpu.sync_copy(sbuf, hist_sh.at[pl.ds(bbase, BINS_T)])

        @pl.when(sub == 0)
        def _():
            totb[...] = jnp.full((16,), DUMP_PACK, jnp.int32)
            pltpu.sync_copy(totb, hist_sh.at[pl.ds(SENTINEL, 16)])
        plsc.subcore_barrier()

        def p3_issue(i, par):
            ck = sub + 16 * i
            cond = (ck < NCH) | ((i == CPT) & (sub == 0))

            @pl.when(cond)
            def _():
                pltpu.async_copy(hist_sh.at[ids_v.at[i]], pb2.at[par],
                                 psm.at[par])

        p3_issue(0, 0)

        def p3_chunk(i, mt):
            par = i & 1
            ck = sub + 16 * i
            cond = (ck < NCH) | ((i == CPT) & (sub == 0))
            cke = jnp.where(i >= CPT, NCH, ck)

            def do(mt):
                pltpu.make_async_copy(hist_sh.at[ids_v.at[i]], pb2.at[par],
                                      psm.at[par]).wait()
                for g in range(CP // 16):
                    pv = pb2[par, pl.ds(g * 16, 16)]
                    sl2[0, pl.ds(g * 16, 16)] = pv >> 5
                    cvals[pl.ds(g * 16, 16)] = (
                        (ids_v[i, pl.ds(g * 16, 16)] << 5) | (pv & 31))
                pltpu.sync_copy(cvals, cv_sh.at[sl2.at[0]])
                for g in range(CP // 16):
                    pv = pb2[par, pl.ds(g * 16, 16)]
                    valid = pv < DUMP_PACK
                    occ = jnp.where(valid, 1, 0).astype(jnp.int32)
                    x = occ
                    for k in (1, 2, 4, 8):
                        sh = x.at[jnp.maximum(lanes - k, 0)].get(
                            mode="promise_in_bounds")
                        x = x + jnp.where(lanes >= k, sh, 0)
                    excl = x - occ
                    pos = jnp.where(valid, sub * REG + mt + excl, DUMPCELL)
                    posb[0, pl.ds(g * 16, 16)] = pos
                    gidx = b * N + cke * CP + g * 16 + lanes
                    cvals[pl.ds(g * 16, 16)] = (
                        gidx * 2048 + (pv >> 5))
                    mt = mt + x[15]
                pltpu.sync_copy(cvals, list_sh.at[posb.at[0]])
                return mt

            return lax.cond(cond, do, lambda m: m, mt)

        def p3_loop(i, mt):
            mt2 = p3_chunk(i, mt)
            p3_issue(i + 1, 1 - (i & 1))
            return mt2
        mt = lax.fori_loop(0, CPT + 1, p3_loop, jnp.int32(0))

        def p3b(j, _):
            pltpu.sync_copy(list_sh.at[pl.ds(sub * REG + j * 32, 32)], lbuf)
            for g in range(2):
                v = lbuf[pl.ds(g * 16, 16)]
                sidx[0, pl.ds(g * 16, 16)] = v & 2047
                gb[0, pl.ds(g * 16, 16)] = v >> 11
            pltpu.sync_copy(rows_hbm.at[gb.at[0]], rv)
            pltpu.sync_copy(rv, acc_sh.at[sidx.at[0]], add=True)
            return 0
        n32 = (mt + 31) >> 5
        lax.fori_loop(0, n32, p3b, 0)
        plsc.subcore_barrier()

        r0 = sub * CP
        pltpu.sync_copy(acc_sh.at[pl.ds(r0, CP), :], fv1)
        pltpu.sync_copy(cv_sh.at[pl.ds(r0, CP)], cvals)

        def frow(g, _):
            cv = (cvals[pl.ds(g * 16, 16)] & 31).astype(jnp.float32)
            cf = jnp.maximum(cv, jnp.float32(1.0))
            scale16 = jnp.float32(MAXP) / (cf * cf)
            for rl in range(16):
                r = g * 16 + rl
                scale = scale16[rl]
                for off in KOFFS:
                    ofeat[r, pl.ds(off, 16)] = (
                        fv1[r, pl.ds(off, 16)] * scale)
            return 0
        lax.fori_loop(0, CP // 16, frow, 0)
        pltpu.sync_copy(ofeat, enc_hbm.at[b, pl.ds(r0, CP), :])

        for g in range(CP // 16):
            cvv = cvals[pl.ds(g * 16, 16)]
            vv = cvv >> 5
            cnt = cvv & 31
            zc = (vv.astype(jnp.float32) / jnp.float32(NY * NX)).astype(
                jnp.int32)
            rem = vv - zc * (NY * NX)
            yc = (rem.astype(jnp.float32) / jnp.float32(NX)).astype(jnp.int32)
            xc = rem - yc * NX
            cobuf[pl.ds(g * 16, 16)] = zc
            cobuf[pl.ds(CP + g * 16, 16)] = yc
            cobuf[pl.ds(2 * CP + g * 16, 16)] = xc
            cobuf[pl.ds(3 * CP + g * 16, 16)] = cnt
        for p in range(4):
            pltpu.sync_copy(cobuf.at[pl.ds(p * CP, CP)],
                            cc_hbm.at[pl.ds((b * 4 + p) * VP + r0, CP)])
        plsc.subcore_barrier()
        return 0

    lax.fori_loop(0, B // 2, one_batch, 0)


@jax.jit
def kernel(point_cloud_features):
    mesh = plsc.VectorSubcoreMesh(core_axis_name="c", subcore_axis_name="s")
    run = functools.partial(
        pl.kernel,
        out_type=[
            jax.ShapeDtypeStruct((B, VP, CW), jnp.float32),
            jax.ShapeDtypeStruct((B * VP * 4,), jnp.int32),
        ],
        mesh=mesh,
        compiler_params=pltpu.CompilerParams(use_tc_tiling_on_sc=False),
        scratch_types=[
            pltpu.VMEM_SHARED((HIST_SZ,), jnp.int32),
            pltpu.VMEM_SHARED((VP, CW), jnp.float32),
            pltpu.VMEM_SHARED((VP,), jnp.int32),
            pltpu.VMEM_SHARED((256,), jnp.int32),
            pltpu.VMEM_SHARED((LIST_SZ,), jnp.int32),
            pltpu.VMEM((CP, CW), jnp.float32),
            pltpu.VMEM((VP,), jnp.int32),
            pltpu.VMEM((CP, CW), jnp.float32),
            pltpu.VMEM((BINS_T,), jnp.int32),
            pltpu.VMEM((BINS_T,), jnp.int32),
            pltpu.VMEM((CPT + 1, CP), jnp.int32),
            pltpu.VMEM((2, CP), jnp.int32),
            pltpu.VMEM((1, CP), jnp.int32),
            pltpu.VMEM((CP,), jnp.int32),
            pltpu.VMEM((16,), jnp.int32),
            pltpu.VMEM((256,), jnp.int32),
            pltpu.VMEM((2, CP), jnp.int32),
            pltpu.VMEM((2, CP), jnp.int32),
            pltpu.VMEM((2, CP), jnp.int32),
            pltpu.VMEM((2, CP), jnp.float32),
            pltpu.VMEM((2, CP), jnp.float32),
            pltpu.VMEM((2, CP), jnp.float32),
            pltpu.VMEM((CP * 4,), jnp.int32),
            pltpu.VMEM((CP, CW), jnp.float32),
            pltpu.VMEM((VP,), jnp.int32),
            pltpu.VMEM((1, CP), jnp.int32),
            pltpu.VMEM((32,), jnp.int32),
            pltpu.VMEM((1, 32), jnp.int32),
            pltpu.VMEM((1, 32), jnp.int32),
            pltpu.VMEM((32, CW), jnp.float32),
            pltpu.SemaphoreType.DMA,
            pltpu.SemaphoreType.DMA((2,)),
            pltpu.SemaphoreType.DMA((2,)),
            pltpu.SemaphoreType.DMA((2,)),
            pltpu.SemaphoreType.DMA((2,)),
            pltpu.SemaphoreType.DMA((2,)),
        ],
    )(_sc_kernel)
    pcs_p = jnp.pad(point_cloud_features, ((0, 0), (0, 0), (0, CW - C)))
    rows2d = pcs_p.reshape(B * N, CW)
    xyzf = point_cloud_features[:, :, :3].reshape(-1)
    enc_p, cc_p = run(rows2d, xyzf)
    cc = cc_p.reshape(B, 4, VP)
    coords = jnp.stack([cc[:, 0, :MAXV], cc[:, 1, :MAXV], cc[:, 2, :MAXV]],
                       axis=2)
    return (enc_p[:, :MAXV, :C], coords, cc[:, 3, :MAXV])


# speedup vs baseline: 4.9793x; 1.0010x over previous
import functools

import jax
import jax.numpy as jnp
from jax import lax
from jax.experimental import pallas as pl
from jax.experimental.pallas import tpu as pltpu
from jax.experimental.pallas import tpu_sc as plsc

B = 4
N = 50000
C = 101
NX, NY, NZ = 40, 80, 80
VSIZE = 0.05
MAXV = 2000
MAXP = 25
SENTINEL = NX * NY * NZ

CP = 128
NCH = N // CP
CPT = (NCH + 15) // 16
TAIL = N - NCH * CP
CW = 104
KOFFS = (0, 16, 32, 48, 64, 80, 88)
VP = 2048
BINS_T = SENTINEL // 16
HIST_SZ = SENTINEL + 16
DUMP = MAXV
DUMP_PACK = DUMP * 32
REG = (CPT + 1) * CP
LIST_SZ = 16 * REG + 16
DUMPCELL = 16 * REG


def _ids_group(x, y, z, nvalid):
    lox, loy, loz = jnp.float32(-1.0), jnp.float32(-2.0), jnp.float32(-2.0)
    hix, hiy, hiz = jnp.float32(1.0), jnp.float32(2.0), jnp.float32(2.0)
    vs = jnp.float32(VSIZE)
    tx = ((x - lox) / vs).astype(jnp.int32)
    ty = ((y - loy) / vs).astype(jnp.int32)
    tz = ((z - loz) / vs).astype(jnp.int32)
    inr = ((x >= lox) & (x < hix) & (y >= loy) & (y < hiy)
           & (z >= loz) & (z < hiz)
           & (tx >= 0) & (tx < NX) & (ty >= 0) & (ty < NY)
           & (tz >= 0) & (tz < NZ))
    if nvalid < 16:
        inr = inr & (lax.iota(jnp.int32, 16) < nvalid)
    flat = tz * (NY * NX) + ty * NX + tx
    return jnp.where(inr, flat, SENTINEL)


def _sc_kernel(rows_hbm, flat_hbm, enc_hbm, cc_hbm,
               hist_sh, acc_sh, cv_sh, tot_sh, list_sh,
               zbuf, zibuf, fv1, hbuf, sbuf, ids_v, pb2, sl2, cvals,
               totb, tbuf, xidx, yidx, zidx, xb2, yb2, zb2, cobuf, ofeat,
               padc, posb, lbuf, gb, sidx, rv,
               sem, gsx, gsy, gsz, psm, fsm, hsm):
    core = lax.axis_index("c")
    sub = lax.axis_index("s")
    lanes = lax.iota(jnp.int32, 16)

    def zloop(r, _):
        for off in KOFFS:
            zbuf[r, pl.ds(off, 16)] = jnp.zeros((16,), jnp.float32)
        return 0
    lax.fori_loop(0, CP, zloop, 0)

    def ziloop(i, _):
        zibuf[pl.ds(i * 16, 16)] = jnp.zeros((16,), jnp.int32)
        return 0
    lax.fori_loop(0, VP // 16, ziloop, 0)

    def one_batch(bb, _):
        b = 2 * core + bb

        pltpu.sync_copy(zbuf, acc_sh.at[pl.ds(sub * CP, CP), :])
        for q in range(BINS_T // VP):
            pltpu.sync_copy(zibuf,
                            hist_sh.at[pl.ds(sub * BINS_T + q * VP, VP)])
        rem0 = BINS_T - (BINS_T // VP) * VP
        if rem0:
            pltpu.sync_copy(
                zibuf.at[pl.ds(0, rem0)],
                hist_sh.at[pl.ds(sub * BINS_T + BINS_T - rem0, rem0)])

        @pl.when(sub == 0)
        def _():
            pltpu.sync_copy(zibuf, cv_sh)
            pltpu.sync_copy(zibuf.at[pl.ds(0, 16)],
                            hist_sh.at[pl.ds(SENTINEL, 16)])

        def oloop(i, _):
            cvals[pl.ds(i * 16, 16)] = jnp.full((16,), 1, jnp.int32)
            return 0
        lax.fori_loop(0, CP // 16, oloop, 0)

        padval = (b * N) * 2048 + DUMP

        def ploop(i, _):
            padc[pl.ds(i * 16, 16)] = jnp.full((16,), padval, jnp.int32)
            return 0
        lax.fori_loop(0, VP // 16, ploop, 0)
        pltpu.sync_copy(padc, list_sh.at[pl.ds(sub * REG, VP)])
        pltpu.sync_copy(padc.at[pl.ds(0, REG - VP)],
                        list_sh.at[pl.ds(sub * REG + VP, REG - VP)])
        plsc.subcore_barrier()

        def p1_issue(i, par):
            ck = sub + 16 * i

            @pl.when(ck < NCH)
            def _():
                fbase = (b * N + ck * CP) * 3
                for g in range(CP // 16):
                    ix = fbase + (g * 16 + lanes) * 3
                    xidx[par, pl.ds(g * 16, 16)] = ix
                    yidx[par, pl.ds(g * 16, 16)] = ix + 1
                    zidx[par, pl.ds(g * 16, 16)] = ix + 2
                pltpu.async_copy(flat_hbm.at[xidx.at[par]], xb2.at[par],
                                 gsx.at[par])
                pltpu.async_copy(flat_hbm.at[yidx.at[par]], yb2.at[par],
                                 gsy.at[par])
                pltpu.async_copy(flat_hbm.at[zidx.at[par]], zb2.at[par],
                                 gsz.at[par])

        p1_issue(0, 0)

        def p1_chunk(i, _):
            par = i & 1
            ck = sub + 16 * i

            @pl.when(ck < NCH)
            def _():
                pltpu.make_async_copy(flat_hbm.at[xidx.at[par]],
                                      xb2.at[par], gsx.at[par]).wait()
                pltpu.make_async_copy(flat_hbm.at[yidx.at[par]],
                                      yb2.at[par], gsy.at[par]).wait()
                pltpu.make_async_copy(flat_hbm.at[zidx.at[par]],
                                      zb2.at[par], gsz.at[par]).wait()
                for g in range(CP // 16):
                    idv = _ids_group(xb2[par, pl.ds(g * 16, 16)],
                                     yb2[par, pl.ds(g * 16, 16)],
                                     zb2[par, pl.ds(g * 16, 16)], 16)
                    ids_v[i, pl.ds(g * 16, 16)] = idv
            p1_issue(i + 1, 1 - par)

            @pl.when(ck < NCH)
            def _():
                @pl.when(i > 0)
                def _():
                    pltpu.make_async_copy(cvals, hist_sh.at[ids_v.at[i]],
                                          hsm).wait()
                pltpu.async_copy(cvals, hist_sh.at[ids_v.at[i]], hsm,
                                 add=True)
            return 0
        lax.fori_loop(0, CPT, p1_chunk, 0)
        pltpu.make_async_copy(cvals, hist_sh.at[ids_v.at[0]], hsm).wait()

        @pl.when(sub == 0)
        def _():
            fbase = (b * N + NCH * CP) * 3
            for g in range(CP // 16):
                p = g * 16 + lanes
                p = jnp.where(p < TAIL, p, 0)
                ix = fbase + p * 3
                xidx[0, pl.ds(g * 16, 16)] = ix
                yidx[0, pl.ds(g * 16, 16)] = ix + 1
                zidx[0, pl.ds(g * 16, 16)] = ix + 2
            cx = pltpu.async_copy(flat_hbm.at[xidx.at[0]], xb2.at[0],
                                  gsx.at[0])
            cy = pltpu.async_copy(flat_hbm.at[yidx.at[0]], yb2.at[0],
                                  gsy.at[0])
            cz = pltpu.async_copy(flat_hbm.at[zidx.at[0]], zb2.at[0],
                                  gsz.at[0])
            cx.wait()
            cy.wait()
            cz.wait()
            for g in range(CP // 16):
                nval = max(0, min(16, TAIL - g * 16))
                idv = _ids_group(xb2[0, pl.ds(g * 16, 16)],
                                 yb2[0, pl.ds(g * 16, 16)],
                                 zb2[0, pl.ds(g * 16, 16)], nval)
                ids_v[CPT, pl.ds(g * 16, 16)] = idv
            pltpu.sync_copy(cvals, hist_sh.at[ids_v.at[CPT]], add=True)
        plsc.subcore_barrier()

        bbase = sub * BINS_T
        pltpu.sync_copy(hist_sh.at[pl.ds(bbase, BINS_T)], hbuf)

        def pass_a(i, acc):
            v = hbuf[pl.ds(i * 16, 16)]
            occ = jnp.where(v > 0, 1, 0).astype(jnp.int32)
            sbuf[pl.ds(i * 16, 16)] = occ
            return acc + occ
        acc16 = lax.fori_loop(0, BINS_T // 16, pass_a,
                              jnp.zeros((16,), jnp.int32))
        total = jnp.int32(0)
        for l in range(16):
            total = total + acc16[l]

        totb[...] = jnp.full((16,), total, jnp.int32)
        pltpu.sync_copy(totb, tot_sh.at[pl.ds(sub * 16, 16)])
        plsc.subcore_barrier()
        pltpu.sync_copy(tot_sh, tbuf)
        base = jnp.int32(0)
        for t in range(16):
            tv = tbuf[pl.ds(t * 16, 16)]
            base = base + jnp.where(t < sub, tv[0], 0)

        @pl.when(base < MAXV)
        def _():
            def pass_b(i, carry):
                live = carry + base < MAXV

                def slow(c):
                    v = hbuf[pl.ds(i * 16, 16)]
                    occ = sbuf[pl.ds(i * 16, 16)]
                    x = occ
                    for k in (1, 2, 4, 8):
                        sh = x.at[jnp.maximum(lanes - k, 0)].get(
                            mode="promise_in_bounds")
                        x = x + jnp.where(lanes >= k, sh, 0)
                    inc = x + c
                    slot = inc - occ + base
                    valid = (occ > 0) & (slot < MAXV)
                    packed = jnp.where(valid,
                                       slot * 32 + jnp.minimum(v, MAXP),
                                       DUMP_PACK)
                    sbuf[pl.ds(i * 16, 16)] = packed
                    return inc[15]

                def fast(c):
                    sbuf[pl.ds(i * 16, 16)] = jnp.full((16,), DUMP_PACK,
                                                       jnp.int32)
                    return c
                return lax.cond(live, slow, fast, carry)
            lax.fori_loop(0, BINS_T // 16, pass_b, jnp.int32(0))

        @pl.when(base >= MAXV)
        def _():
            def dump_fill(i, _):
                sbuf[pl.ds(i * 16, 16)] = jnp.full((16,), DUMP_PACK,
                                                   jnp.int32)
                return 0
            lax.fori_loop(0, BINS_T // 16, dump_fill, 0)
        pltpu.sync_copy(sbuf, hist_sh.at[pl.ds(bbase, BINS_T)])

        @pl.when(sub == 0)
        def _():
            totb[...] = jnp.full((16,), DUMP_PACK, jnp.int32)
            pltpu.sync_copy(totb, hist_sh.at[pl.ds(SENTINEL, 16)])
        plsc.subcore_barrier()

        def p3_issue(i, par):
            ck = sub + 16 * i
            cond = (ck < NCH) | ((i == CPT) & (sub == 0))

            @pl.when(cond)
            def _():
                pltpu.async_copy(hist_sh.at[ids_v.at[i]], pb2.at[par],
                                 psm.at[par])

        p3_issue(0, 0)

        def p3_chunk(i, mt):
            par = i & 1
            ck = sub + 16 * i
            cond = (ck < NCH) | ((i == CPT) & (sub == 0))
            cke = jnp.where(i >= CPT, NCH, ck)

            def do(mt):
                pltpu.make_async_copy(hist_sh.at[ids_v.at[i]], pb2.at[par],
                                      psm.at[par]).wait()
                for g in range(CP // 16):
                    pv = pb2[par, pl.ds(g * 16, 16)]
                    sl2[0, pl.ds(g * 16, 16)] = pv >> 5
                    cvals[pl.ds(g * 16, 16)] = (
                        (ids_v[i, pl.ds(g * 16, 16)] << 5) | (pv & 31))
                pltpu.sync_copy(cvals, cv_sh.at[sl2.at[0]])
                for g in range(CP // 16):
                    pv = pb2[par, pl.ds(g * 16, 16)]
                    valid = pv < DUMP_PACK
                    occ = jnp.where(valid, 1, 0).astype(jnp.int32)
                    x = occ
                    for k in (1, 2, 4, 8):
                        sh = x.at[jnp.maximum(lanes - k, 0)].get(
                            mode="promise_in_bounds")
                        x = x + jnp.where(lanes >= k, sh, 0)
                    excl = x - occ
                    pos = jnp.where(valid, sub * REG + mt + excl, DUMPCELL)
                    posb[0, pl.ds(g * 16, 16)] = pos
                    gidx = b * N + cke * CP + g * 16 + lanes
                    cvals[pl.ds(g * 16, 16)] = (
                        gidx * 2048 + (pv >> 5))
                    mt = mt + x[15]
                pltpu.sync_copy(cvals, list_sh.at[posb.at[0]])
                return mt

            return lax.cond(cond, do, lambda m: m, mt)

        def p3_loop(i, mt):
            mt2 = p3_chunk(i, mt)
            p3_issue(i + 1, 1 - (i & 1))
            return mt2
        mt = lax.fori_loop(0, CPT + 1, p3_loop, jnp.int32(0))

        def p3b(j, _):
            pltpu.sync_copy(list_sh.at[pl.ds(sub * REG + j * 32, 32)], lbuf)
            for g in range(2):
                v = lbuf[pl.ds(g * 16, 16)]
                sidx[0, pl.ds(g * 16, 16)] = v & 2047
                gb[0, pl.ds(g * 16, 16)] = v >> 11
            pltpu.sync_copy(rows_hbm.at[gb.at[0]], rv)
            pltpu.sync_copy(rv, acc_sh.at[sidx.at[0]], add=True)
            return 0
        n32 = (mt + 31) >> 5
        lax.fori_loop(0, n32, p3b, 0)
        plsc.subcore_barrier()

        r0 = sub * CP
        pltpu.sync_copy(acc_sh.at[pl.ds(r0, CP), :], fv1)
        pltpu.sync_copy(cv_sh.at[pl.ds(r0, CP)], cvals)

        def frow(g, _):
            cv = (cvals[pl.ds(g * 16, 16)] & 31).astype(jnp.float32)
            cf = jnp.maximum(cv, jnp.float32(1.0))
            scale16 = jnp.float32(MAXP) / (cf * cf)
            for rl in range(16):
                r = g * 16 + rl
                scale = scale16[rl]
                for off in KOFFS:
                    ofeat[r, pl.ds(off, 16)] = (
                        fv1[r, pl.ds(off, 16)] * scale)
            return 0
        lax.fori_loop(0, CP // 16, frow, 0)
        pltpu.sync_copy(ofeat, enc_hbm.at[b, pl.ds(r0, CP), :])

        for g in range(CP // 16):
            cvv = cvals[pl.ds(g * 16, 16)]
            vv = cvv >> 5
            cnt = cvv & 31
            zc = (vv.astype(jnp.float32) / jnp.float32(NY * NX)).astype(
                jnp.int32)
            rem = vv - zc * (NY * NX)
            yc = (rem.astype(jnp.float32) / jnp.float32(NX)).astype(jnp.int32)
            xc = rem - yc * NX
            cobuf[pl.ds(g * 16, 16)] = zc
            cobuf[pl.ds(CP + g * 16, 16)] = yc
            cobuf[pl.ds(2 * CP + g * 16, 16)] = xc
            cobuf[pl.ds(3 * CP + g * 16, 16)] = cnt
        for p in range(4):
            pltpu.sync_copy(cobuf.at[pl.ds(p * CP, CP)],
                            cc_hbm.at[pl.ds((b * 4 + p) * VP + r0, CP)])
        plsc.subcore_barrier()
        return 0

    lax.fori_loop(0, B // 2, one_batch, 0)


@jax.jit
def kernel(point_cloud_features):
    mesh = plsc.VectorSubcoreMesh(core_axis_name="c", subcore_axis_name="s")
    run = functools.partial(
        pl.kernel,
        out_type=[
            jax.ShapeDtypeStruct((B, VP, CW), jnp.float32),
            jax.ShapeDtypeStruct((B * VP * 4,), jnp.int32),
        ],
        mesh=mesh,
        compiler_params=pltpu.CompilerParams(use_tc_tiling_on_sc=False),
        scratch_types=[
            pltpu.VMEM_SHARED((HIST_SZ,), jnp.int32),
            pltpu.VMEM_SHARED((VP, CW), jnp.float32),
            pltpu.VMEM_SHARED((VP,), jnp.int32),
            pltpu.VMEM_SHARED((256,), jnp.int32),
            pltpu.VMEM_SHARED((LIST_SZ,), jnp.int32),
            pltpu.VMEM((CP, CW), jnp.float32),
            pltpu.VMEM((VP,), jnp.int32),
            pltpu.VMEM((CP, CW), jnp.float32),
            pltpu.VMEM((BINS_T,), jnp.int32),
            pltpu.VMEM((BINS_T,), jnp.int32),
            pltpu.VMEM((CPT + 1, CP), jnp.int32),
            pltpu.VMEM((2, CP), jnp.int32),
            pltpu.VMEM((1, CP), jnp.int32),
            pltpu.VMEM((CP,), jnp.int32),
            pltpu.VMEM((16,), jnp.int32),
            pltpu.VMEM((256,), jnp.int32),
            pltpu.VMEM((2, CP), jnp.int32),
            pltpu.VMEM((2, CP), jnp.int32),
            pltpu.VMEM((2, CP), jnp.int32),
            pltpu.VMEM((2, CP), jnp.float32),
            pltpu.VMEM((2, CP), jnp.float32),
            pltpu.VMEM((2, CP), jnp.float32),
            pltpu.VMEM((CP * 4,), jnp.int32),
            pltpu.VMEM((CP, CW), jnp.float32),
            pltpu.VMEM((VP,), jnp.int32),
            pltpu.VMEM((1, CP), jnp.int32),
            pltpu.VMEM((32,), jnp.int32),
            pltpu.VMEM((1, 32), jnp.int32),
            pltpu.VMEM((1, 32), jnp.int32),
            pltpu.VMEM((32, CW), jnp.float32),
            pltpu.SemaphoreType.DMA,
            pltpu.SemaphoreType.DMA((2,)),
            pltpu.SemaphoreType.DMA((2,)),
            pltpu.SemaphoreType.DMA((2,)),
            pltpu.SemaphoreType.DMA((2,)),
            pltpu.SemaphoreType.DMA((2,)),
            pltpu.SemaphoreType.DMA,
        ],
    )(_sc_kernel)
    pcs_p = jnp.pad(point_cloud_features, ((0, 0), (0, 0), (0, CW - C)))
    rows2d = pcs_p.reshape(B * N, CW)
    xyzf = point_cloud_features[:, :, :3].reshape(-1)
    enc_p, cc_p = run(rows2d, xyzf)
    cc = cc_p.reshape(B, 4, VP)
    coords = jnp.stack([cc[:, 0, :MAXV], cc[:, 1, :MAXV], cc[:, 2, :MAXV]],
                       axis=2)
    return (enc_p[:, :MAXV, :C], coords, cc[:, 3, :MAXV])
